# baseline (speedup 1.0000x reference)
import functools

import jax
import jax.numpy as jnp
from jax import lax
from jax.experimental import pallas as pl
from jax.experimental.pallas import tpu as pltpu
from jax.experimental.pallas import tpu_sc as plsc

B = 16384
MF_DIM = 16
MLP_HALF = 32

_NC, _NS = 2, 16
_NW = _NC * _NS
_BPW = B // _NW


def _sc_gather_body(user_hbm, item_hbm, mfu_t, mfi_t, mlpu_t, mlpi_t,
                    out_mfu, out_mfi, out_mlpu, out_mlpi,
                    idx_u, idx_i, mfu_v, mfi_v, mlpu_v, mlpi_v, sem):
    wid = lax.axis_index("s") * _NC + lax.axis_index("c")
    base = wid * _BPW
    pltpu.sync_copy(user_hbm.at[pl.ds(base, _BPW)], idx_u)
    pltpu.sync_copy(item_hbm.at[pl.ds(base, _BPW)], idx_i)
    cp1 = pltpu.async_copy(mfu_t.at[idx_u], mfu_v, sem)
    cp2 = pltpu.async_copy(mfi_t.at[idx_i], mfi_v, sem)
    cp3 = pltpu.async_copy(mlpu_t.at[idx_u], mlpu_v, sem)
    cp4 = pltpu.async_copy(mlpi_t.at[idx_i], mlpi_v, sem)
    cp1.wait()
    cp2.wait()
    cp3.wait()
    cp4.wait()
    pltpu.sync_copy(mfu_v, out_mfu.at[pl.ds(base, _BPW)])
    pltpu.sync_copy(mfi_v, out_mfi.at[pl.ds(base, _BPW)])
    pltpu.sync_copy(mlpu_v, out_mlpu.at[pl.ds(base, _BPW)])
    pltpu.sync_copy(mlpi_v, out_mlpi.at[pl.ds(base, _BPW)])


@functools.cache
def _sc_gather():
    return pl.kernel(
        _sc_gather_body,
        out_type=[
            jax.ShapeDtypeStruct((B, MF_DIM), jnp.float32),
            jax.ShapeDtypeStruct((B, MF_DIM), jnp.float32),
            jax.ShapeDtypeStruct((B, MLP_HALF), jnp.float32),
            jax.ShapeDtypeStruct((B, MLP_HALF), jnp.float32),
        ],
        mesh=plsc.VectorSubcoreMesh(core_axis_name="c", subcore_axis_name="s"),
        compiler_params=pltpu.CompilerParams(use_tc_tiling_on_sc=False),
        scratch_types=[
            pltpu.VMEM((_BPW,), jnp.int32),
            pltpu.VMEM((_BPW,), jnp.int32),
            pltpu.VMEM((_BPW, MF_DIM), jnp.float32),
            pltpu.VMEM((_BPW, MF_DIM), jnp.float32),
            pltpu.VMEM((_BPW, MLP_HALF), jnp.float32),
            pltpu.VMEM((_BPW, MLP_HALF), jnp.float32),
            pltpu.SemaphoreType.DMA,
        ],
    )


def _tc_mlp_body(mfu_ref, mfi_ref, mlpu_ref, mlpi_ref,
                 W1u_ref, W1i_ref, b1_ref, W2_ref, b2_ref,
                 Wfa_ref, Wfb_ref, bf_ref, out_ref):
    xmf = mfu_ref[...] * mfi_ref[...]
    h1 = mlpu_ref[...] @ W1u_ref[...] + mlpi_ref[...] @ W1i_ref[...] + b1_ref[...]
    h1 = jnp.maximum(h1, 0.0)
    h2 = jnp.maximum(h1 @ W2_ref[...] + b2_ref[...], 0.0)
    out_ref[...] = xmf @ Wfa_ref[...] + h2 @ Wfb_ref[...] + bf_ref[0, 0]


def _tc_mlp(mfu, mfi, mlpu, mlpi, W1, b1, W2, b2, Wf, bf):
    blk = 2048
    grid = (B // blk,)
    W1u = W1[:MLP_HALF]
    W1i = W1[MLP_HALF:]
    Wfa = Wf[:MF_DIM]
    Wfb = Wf[MF_DIM:]
    row = lambda i: (i, 0)
    rep = lambda i: (0, 0)
    return pl.pallas_call(
        _tc_mlp_body,
        grid=grid,
        in_specs=[
            pl.BlockSpec((blk, MF_DIM), row),
            pl.BlockSpec((blk, MF_DIM), row),
            pl.BlockSpec((blk, MLP_HALF), row),
            pl.BlockSpec((blk, MLP_HALF), row),
            pl.BlockSpec((MLP_HALF, 32), rep),
            pl.BlockSpec((MLP_HALF, 32), rep),
            pl.BlockSpec((1, 32), rep),
            pl.BlockSpec((32, 16), rep),
            pl.BlockSpec((1, 16), rep),
            pl.BlockSpec((MF_DIM, 1), rep),
            pl.BlockSpec((16, 1), rep),
            pl.BlockSpec((1, 1), rep),
        ],
        out_specs=pl.BlockSpec((blk, 1), row),
        out_shape=jax.ShapeDtypeStruct((B, 1), jnp.float32),
    )(mfu, mfi, mlpu, mlpi, W1u, W1i, b1.reshape(1, -1), W2,
      b2.reshape(1, -1), Wfa, Wfb, bf.reshape(1, 1))


def kernel(user, item, mf_user_embed, mf_item_embed, mlp_user_embed,
           mlp_item_embed, W1, b1, W2, b2, Wf, bf):
    mfu, mfi, mlpu, mlpi = _sc_gather()(
        user.astype(jnp.int32), item.astype(jnp.int32),
        mf_user_embed, mf_item_embed, mlp_user_embed, mlp_item_embed)
    return _tc_mlp(mfu, mfi, mlpu, mlpi, W1, b1, W2, b2, Wf, bf)
